# Initial kernel scaffold; baseline (speedup 1.0000x reference)
import jax
import jax.numpy as jnp
from jax.experimental import pallas as pl


def kernel(x, edge_index, W1n, W1s, b1, Wl, bl, W2n, W2s, b2):
    raise NotImplementedError("write your pallas kernel here")


# speedup vs baseline: 5.3938x; 5.3938x over previous
import functools

import jax
import jax.numpy as jnp
from jax import lax
from jax.experimental import pallas as pl
from jax.experimental.pallas import tpu as pltpu
from jax.experimental.pallas import tpu_sc as plsc

N = 10000
D = 128
E = 320000

NC = 2
NS = 16
NW = NC * NS
EPT = E // NW
K = 80
NCHUNK = EPT // K
RPT = 624
ZR = 104

_mesh = plsc.VectorSubcoreMesh(core_axis_name="c", subcore_axis_name="s")


def _fill(rows_v, val):
    def f(k, _):
        rows_v[k // 8, pl.ds((k % 8) * 16, 16)] = jnp.full((16,), val,
                                                           jnp.float32)
        return 0
    lax.fori_loop(0, K * 8, f, 0)


def _zero_acc(s, rows_v, agg_s):
    r0 = s * RPT
    for t in range(RPT // K):
        pltpu.sync_copy(rows_v, agg_s.at[pl.ds(r0 + t * K, K)])
    rem = RPT - (RPT // K) * K
    if rem:
        pltpu.sync_copy(rows_v.at[pl.ds(0, rem)],
                        agg_s.at[pl.ds(r0 + RPT - rem, rem)])

    @pl.when(s == NS - 1)
    def _():
        pltpu.sync_copy(rows_v.at[pl.ds(0, 16)], agg_s.at[pl.ds(N - 16, 16)])


def _copy_acc_out(c, s, agg_s, dest):
    r0 = s * RPT
    pltpu.sync_copy(agg_s.at[pl.ds(r0, RPT)], dest.at[c, pl.ds(r0, RPT)])

    @pl.when(s == NS - 1)
    def _():
        pltpu.sync_copy(agg_s.at[pl.ds(N - 16, 16)],
                        dest.at[c, pl.ds(N - 16, 16)])


def _seg_body(x_hbm, src_hbm, dst_hbm, agg_out,
              src_v, dst_v, rows_v, agg_s, sem, deg_out=None):
    c = lax.axis_index("c")
    s = lax.axis_index("s")
    base0 = (c * NS + s) * EPT

    if deg_out is not None:
        _fill(rows_v, 0.0)
        _zero_acc(s, rows_v, agg_s)
        _fill(rows_v, 1.0)
        plsc.subcore_barrier()

        def dchunk(i, _):
            pltpu.sync_copy(dst_hbm.at[pl.ds(base0 + i * K, K)], dst_v)
            pltpu.sync_copy(rows_v, agg_s.at[dst_v], add=True)
            return 0
        lax.fori_loop(0, NCHUNK, dchunk, 0)
        plsc.subcore_barrier()
        _copy_acc_out(c, s, agg_s, deg_out)

    _fill(rows_v, 0.0)
    _zero_acc(s, rows_v, agg_s)
    plsc.subcore_barrier()

    def chunk(i, _):
        base = base0 + i * K
        pltpu.sync_copy(src_hbm.at[pl.ds(base, K)], src_v)
        pltpu.sync_copy(dst_hbm.at[pl.ds(base, K)], dst_v)
        pltpu.async_copy(x_hbm.at[src_v], rows_v, sem).wait()
        pltpu.sync_copy(rows_v, agg_s.at[dst_v], add=True)
        return 0
    lax.fori_loop(0, NCHUNK, chunk, 0)

    plsc.subcore_barrier()
    _copy_acc_out(c, s, agg_s, agg_out)


_SEG_SCRATCH = [
    pltpu.VMEM((K,), jnp.int32),
    pltpu.VMEM((K,), jnp.int32),
    pltpu.VMEM((K, D), jnp.float32),
    pltpu.VMEM_SHARED((N, D), jnp.float32),
    pltpu.SemaphoreType.DMA,
]


@functools.partial(
    pl.kernel,
    out_type=jax.ShapeDtypeStruct((NC, N, D), jnp.float32),
    mesh=_mesh,
    scratch_types=list(_SEG_SCRATCH),
)
def _segsum_sc(x_hbm, src_hbm, dst_hbm, agg_out, *scratch):
    _seg_body(x_hbm, src_hbm, dst_hbm, agg_out, *scratch)


@functools.partial(
    pl.kernel,
    out_type=[jax.ShapeDtypeStruct((NC, N, D), jnp.float32),
              jax.ShapeDtypeStruct((NC, N, D), jnp.float32)],
    mesh=_mesh,
    scratch_types=list(_SEG_SCRATCH),
)
def _segsum_deg_sc(x_hbm, src_hbm, dst_hbm, agg_out, deg_out, *scratch):
    src_v, dst_v, rows_v, agg_s, sem = scratch
    _seg_body(x_hbm, src_hbm, dst_hbm, agg_out,
              src_v, dst_v, rows_v, agg_s, sem, deg_out=deg_out)


BN = 400


def _mid_body(x, a0, a1, d0, d1, w1s, wl, w1n, b1, bl, w2n, w2s, b2,
              y_out, zs_out):
    deg = jnp.maximum(d0[...] + d1[...], 1.0)[:, 0:1]
    mean1 = (a0[...] + a1[...]) / deg
    h = jnp.dot(mean1, w1n[...], preferred_element_type=jnp.float32)
    h += jnp.dot(x[...], w1s[...] + wl[...], preferred_element_type=jnp.float32)
    z = jnp.maximum(h + (b1[...] + bl[...]), 0.0)
    y_out[...] = jnp.dot(z, w2n[...], preferred_element_type=jnp.float32)
    zs_out[...] = (jnp.dot(z, w2s[...], preferred_element_type=jnp.float32)
                   + b2[...])


def _mid_tc(x, a0, a1, d0, d1, W1s, Wl, W1n, b1, bl, W2n, W2s, b2):
    HID = W1n.shape[1]
    LAT = W2n.shape[1]
    grid = N // BN
    row = lambda i: (i, 0)
    rep = lambda i: (0, 0)
    return pl.pallas_call(
        _mid_body,
        grid=(grid,),
        in_specs=[
            pl.BlockSpec((BN, D), row),
            pl.BlockSpec((BN, D), row),
            pl.BlockSpec((BN, D), row),
            pl.BlockSpec((BN, D), row),
            pl.BlockSpec((BN, D), row),
            pl.BlockSpec((D, HID), rep),
            pl.BlockSpec((D, HID), rep),
            pl.BlockSpec((D, HID), rep),
            pl.BlockSpec((1, HID), rep),
            pl.BlockSpec((1, HID), rep),
            pl.BlockSpec((HID, LAT), rep),
            pl.BlockSpec((HID, LAT), rep),
            pl.BlockSpec((1, LAT), rep),
        ],
        out_specs=[
            pl.BlockSpec((BN, LAT), row),
            pl.BlockSpec((BN, LAT), row),
        ],
        out_shape=[
            jax.ShapeDtypeStruct((N, LAT), jnp.float32),
            jax.ShapeDtypeStruct((N, LAT), jnp.float32),
        ],
    )(x, a0, a1, d0, d1, W1s, Wl, W1n, b1, bl, W2n, W2s, b2)


def _fin_body(a0, a1, d0, d1, zs, out):
    deg = jnp.maximum(d0[...] + d1[...], 1.0)[:, 0:1]
    out[...] = (a0[...] + a1[...]) / deg + zs[...]


def _fin_tc(a0, a1, d0, d1, zs):
    grid = N // BN
    row = lambda i: (i, 0)
    return pl.pallas_call(
        _fin_body,
        grid=(grid,),
        in_specs=[
            pl.BlockSpec((BN, D), row),
            pl.BlockSpec((BN, D), row),
            pl.BlockSpec((BN, D), row),
            pl.BlockSpec((BN, D), row),
            pl.BlockSpec((BN, D), row),
        ],
        out_specs=pl.BlockSpec((BN, D), row),
        out_shape=jax.ShapeDtypeStruct((N, D), jnp.float32),
    )(a0, a1, d0, d1, zs)


def kernel(x, edge_index, W1n, W1s, b1, Wl, bl, W2n, W2s, b2):
    src = edge_index[0]
    dst = edge_index[1]
    agg1, degp = _segsum_deg_sc(x, src, dst)
    y, zs = _mid_tc(x, agg1[0], agg1[1], degp[0], degp[1],
                    W1s, Wl, W1n, b1.reshape(1, -1), bl.reshape(1, -1),
                    W2n, W2s, b2.reshape(1, -1))
    agg2 = _segsum_sc(y, src, dst)
    out = _fin_tc(agg2[0], agg2[1], degp[0], degp[1], zs)
    return out


# speedup vs baseline: 8.0705x; 1.4962x over previous
import functools

import jax
import jax.numpy as jnp
from jax import lax
from jax.experimental import pallas as pl
from jax.experimental.pallas import tpu as pltpu
from jax.experimental.pallas import tpu_sc as plsc

N = 10000
D = 128
E = 320000

NC = 2
NS = 16
NW = NC * NS
EPT = E // NW
K = 80
NCHUNK = EPT // K
RPT = 624
ZR = 104

_mesh = plsc.VectorSubcoreMesh(core_axis_name="c", subcore_axis_name="s")


def _fill(buf, val):
    def f(k, _):
        buf[k // 8, pl.ds((k % 8) * 16, 16)] = jnp.full((16,), val,
                                                        jnp.float32)
        return 0
    lax.fori_loop(0, K * 8, f, 0)


def _zero_acc(s, zbuf, agg_s):
    r0 = s * RPT
    for t in range(RPT // K):
        pltpu.sync_copy(zbuf, agg_s.at[pl.ds(r0 + t * K, K)])
    rem = RPT - (RPT // K) * K
    if rem:
        pltpu.sync_copy(zbuf.at[pl.ds(0, rem)],
                        agg_s.at[pl.ds(r0 + RPT - rem, rem)])

    @pl.when(s == NS - 1)
    def _():
        pltpu.sync_copy(zbuf.at[pl.ds(0, 16)], agg_s.at[pl.ds(N - 16, 16)])


def _copy_acc_out(c, s, agg_s, dest):
    r0 = s * RPT
    pltpu.sync_copy(agg_s.at[pl.ds(r0, RPT)], dest.at[c, pl.ds(r0, RPT)])

    @pl.when(s == NS - 1)
    def _():
        pltpu.sync_copy(agg_s.at[pl.ds(N - 16, 16)],
                        dest.at[c, pl.ds(N - 16, 16)])


def _seg_body(x_hbm, src_hbm, dst_hbm, agg_out,
              s0, s1, d0b, d1b, r0b, r1b, agg_s,
              gs0, gs1, ss0, ss1, is0, is1, id0, id1, deg_out=None):
    c = lax.axis_index("c")
    s = lax.axis_index("s")
    base0 = (c * NS + s) * EPT

    def do_pass(gather, dest):
        def super2(u, _):
            base = base0 + (2 * u) * K
            hd0 = pltpu.async_copy(dst_hbm.at[pl.ds(base, K)], d0b, id0)
            hd1 = pltpu.async_copy(dst_hbm.at[pl.ds(base + K, K)], d1b, id1)
            if gather:
                hs0 = pltpu.async_copy(src_hbm.at[pl.ds(base, K)], s0, is0)
                hs1 = pltpu.async_copy(src_hbm.at[pl.ds(base + K, K)], s1,
                                       is1)
                hs0.wait()
                hg0 = pltpu.async_copy(x_hbm.at[s0], r0b, gs0)
                hs1.wait()
                hg1 = pltpu.async_copy(x_hbm.at[s1], r1b, gs1)
                hg0.wait()
                hd0.wait()
                hc0 = pltpu.async_copy(r0b, agg_s.at[d0b], ss0, add=True)
                hg1.wait()
                hd1.wait()
                hc1 = pltpu.async_copy(r1b, agg_s.at[d1b], ss1, add=True)
            else:
                hd0.wait()
                hc0 = pltpu.async_copy(r0b, agg_s.at[d0b], ss0, add=True)
                hd1.wait()
                hc1 = pltpu.async_copy(r0b, agg_s.at[d1b], ss1, add=True)
            hc0.wait()
            hc1.wait()
            return 0
        lax.fori_loop(0, NCHUNK // 2, super2, 0)
        base = base0 + (NCHUNK - 1) * K
        pltpu.sync_copy(dst_hbm.at[pl.ds(base, K)], d0b)
        if gather:
            pltpu.sync_copy(src_hbm.at[pl.ds(base, K)], s0)
            pltpu.async_copy(x_hbm.at[s0], r0b, gs0).wait()
        pltpu.sync_copy(r0b, agg_s.at[d0b], add=True)

        plsc.subcore_barrier()
        _copy_acc_out(c, s, agg_s, dest)

    if deg_out is not None:
        _fill(r1b, 0.0)
        _zero_acc(s, r1b, agg_s)
        _fill(r0b, 1.0)
        plsc.subcore_barrier()
        do_pass(False, deg_out)

    _fill(r1b, 0.0)
    _zero_acc(s, r1b, agg_s)
    plsc.subcore_barrier()
    do_pass(True, agg_out)


_SEG_SCRATCH = [
    pltpu.VMEM((K,), jnp.int32),
    pltpu.VMEM((K,), jnp.int32),
    pltpu.VMEM((K,), jnp.int32),
    pltpu.VMEM((K,), jnp.int32),
    pltpu.VMEM((K, D), jnp.float32),
    pltpu.VMEM((K, D), jnp.float32),
    pltpu.VMEM_SHARED((N, D), jnp.float32),
] + [pltpu.SemaphoreType.DMA] * 8


@functools.partial(
    pl.kernel,
    out_type=jax.ShapeDtypeStruct((NC, N, D), jnp.float32),
    mesh=_mesh,
    scratch_types=list(_SEG_SCRATCH),
)
def _segsum_sc(x_hbm, src_hbm, dst_hbm, agg_out, *scratch):
    _seg_body(x_hbm, src_hbm, dst_hbm, agg_out, *scratch)


@functools.partial(
    pl.kernel,
    out_type=[jax.ShapeDtypeStruct((NC, N, D), jnp.float32),
              jax.ShapeDtypeStruct((NC, N, D), jnp.float32)],
    mesh=_mesh,
    scratch_types=list(_SEG_SCRATCH),
)
def _segsum_deg_sc(x_hbm, src_hbm, dst_hbm, agg_out, deg_out, *scratch):
    _seg_body(x_hbm, src_hbm, dst_hbm, agg_out, *scratch, deg_out=deg_out)


BN = 400


def _mid_body(x, a0, a1, d0, d1, w1s, wl, w1n, b1, bl, w2n, w2s, b2,
              y_out, zs_out):
    deg = jnp.maximum(d0[...] + d1[...], 1.0)[:, 0:1]
    mean1 = (a0[...] + a1[...]) / deg
    h = jnp.dot(mean1, w1n[...], preferred_element_type=jnp.float32)
    h += jnp.dot(x[...], w1s[...] + wl[...], preferred_element_type=jnp.float32)
    z = jnp.maximum(h + (b1[...] + bl[...]), 0.0)
    y_out[...] = jnp.dot(z, w2n[...], preferred_element_type=jnp.float32)
    zs_out[...] = (jnp.dot(z, w2s[...], preferred_element_type=jnp.float32)
                   + b2[...])


def _mid_tc(x, a0, a1, d0, d1, W1s, Wl, W1n, b1, bl, W2n, W2s, b2):
    HID = W1n.shape[1]
    LAT = W2n.shape[1]
    grid = N // BN
    row = lambda i: (i, 0)
    rep = lambda i: (0, 0)
    return pl.pallas_call(
        _mid_body,
        grid=(grid,),
        in_specs=[
            pl.BlockSpec((BN, D), row),
            pl.BlockSpec((BN, D), row),
            pl.BlockSpec((BN, D), row),
            pl.BlockSpec((BN, D), row),
            pl.BlockSpec((BN, D), row),
            pl.BlockSpec((D, HID), rep),
            pl.BlockSpec((D, HID), rep),
            pl.BlockSpec((D, HID), rep),
            pl.BlockSpec((1, HID), rep),
            pl.BlockSpec((1, HID), rep),
            pl.BlockSpec((HID, LAT), rep),
            pl.BlockSpec((HID, LAT), rep),
            pl.BlockSpec((1, LAT), rep),
        ],
        out_specs=[
            pl.BlockSpec((BN, LAT), row),
            pl.BlockSpec((BN, LAT), row),
        ],
        out_shape=[
            jax.ShapeDtypeStruct((N, LAT), jnp.float32),
            jax.ShapeDtypeStruct((N, LAT), jnp.float32),
        ],
    )(x, a0, a1, d0, d1, W1s, Wl, W1n, b1, bl, W2n, W2s, b2)


def _fin_body(a0, a1, d0, d1, zs, out):
    deg = jnp.maximum(d0[...] + d1[...], 1.0)[:, 0:1]
    out[...] = (a0[...] + a1[...]) / deg + zs[...]


def _fin_tc(a0, a1, d0, d1, zs):
    grid = N // BN
    row = lambda i: (i, 0)
    return pl.pallas_call(
        _fin_body,
        grid=(grid,),
        in_specs=[
            pl.BlockSpec((BN, D), row),
            pl.BlockSpec((BN, D), row),
            pl.BlockSpec((BN, D), row),
            pl.BlockSpec((BN, D), row),
            pl.BlockSpec((BN, D), row),
        ],
        out_specs=pl.BlockSpec((BN, D), row),
        out_shape=jax.ShapeDtypeStruct((N, D), jnp.float32),
    )(a0, a1, d0, d1, zs)


def kernel(x, edge_index, W1n, W1s, b1, Wl, bl, W2n, W2s, b2):
    src = edge_index[0]
    dst = edge_index[1]
    agg1, degp = _segsum_deg_sc(x, src, dst)
    y, zs = _mid_tc(x, agg1[0], agg1[1], degp[0], degp[1],
                    W1s, Wl, W1n, b1.reshape(1, -1), bl.reshape(1, -1),
                    W2n, W2s, b2.reshape(1, -1))
    agg2 = _segsum_sc(y, src, dst)
    out = _fin_tc(agg2[0], agg2[1], degp[0], degp[1], zs)
    return out


# speedup vs baseline: 9.2281x; 1.1434x over previous
import functools

import jax
import jax.numpy as jnp
from jax import lax
from jax.experimental import pallas as pl
from jax.experimental.pallas import tpu as pltpu
from jax.experimental.pallas import tpu_sc as plsc

N = 10000
D = 128
E = 320000

NC = 2
NS = 16
NW = NC * NS
EPT = E // NW
K = 80
NCHUNK = EPT // K
RPT = 624
ZR = 104

_mesh = plsc.VectorSubcoreMesh(core_axis_name="c", subcore_axis_name="s")


def _fill(buf, val):
    def f(k, _):
        buf[k // 8, pl.ds((k % 8) * 16, 16)] = jnp.full((16,), val,
                                                        jnp.float32)
        return 0
    lax.fori_loop(0, K * 8, f, 0)


def _zero_acc(s, zbuf, agg_s):
    r0 = s * RPT
    for t in range(RPT // K):
        pltpu.sync_copy(zbuf, agg_s.at[pl.ds(r0 + t * K, K)])
    rem = RPT - (RPT // K) * K
    if rem:
        pltpu.sync_copy(zbuf.at[pl.ds(0, rem)],
                        agg_s.at[pl.ds(r0 + RPT - rem, rem)])

    @pl.when(s == NS - 1)
    def _():
        pltpu.sync_copy(zbuf.at[pl.ds(0, 16)], agg_s.at[pl.ds(N - 16, 16)])


def _copy_acc_out(c, s, agg_s, dest):
    r0 = s * RPT
    pltpu.sync_copy(agg_s.at[pl.ds(r0, RPT)], dest.at[c, pl.ds(r0, RPT)])

    @pl.when(s == NS - 1)
    def _():
        pltpu.sync_copy(agg_s.at[pl.ds(N - 16, 16)],
                        dest.at[c, pl.ds(N - 16, 16)])


NB = 5
BCH = NCHUNK // NB


def _seg_body(x_hbm, src_hbm, dst_hbm, agg_out,
              sb0, sb1, db0, db1, r0b, r1b, agg_s,
              gs0, gs1, ss0, ss1, lb0, lb1, lb2, lb3, deg_out=None):
    c = lax.axis_index("c")
    s = lax.axis_index("s")
    wid = c * NS + s
    sb = (sb0, sb1)
    db = (db0, db1)
    lsem = ((lb0, lb1), (lb2, lb3))

    def do_pass(gather, dest):
        if gather:
            pltpu.async_copy(src_hbm.at[wid, 0], sb[0], lsem[0][0])
        hd = pltpu.async_copy(dst_hbm.at[wid, 0], db[0], lsem[0][1])
        for u in range(NB):
            bb = u & 1
            if gather:
                pltpu.make_async_copy(src_hbm.at[wid, 0], sb[bb],
                                      lsem[bb][0]).wait()
            pltpu.make_async_copy(dst_hbm.at[wid, 0], db[bb],
                                  lsem[bb][1]).wait()
            if u + 1 < NB:
                if gather:
                    pltpu.async_copy(src_hbm.at[wid, u + 1], sb[1 - bb],
                                     lsem[1 - bb][0])
                pltpu.async_copy(dst_hbm.at[wid, u + 1], db[1 - bb],
                                 lsem[1 - bb][1])

            def super2(j, _, bb=bb):
                if gather:
                    hg0 = pltpu.async_copy(x_hbm.at[sb[bb].at[2 * j]], r0b,
                                           gs0)
                    hg1 = pltpu.async_copy(x_hbm.at[sb[bb].at[2 * j + 1]],
                                           r1b, gs1)
                    hg0.wait()
                    hc0 = pltpu.async_copy(r0b, agg_s.at[db[bb].at[2 * j]],
                                           ss0, add=True)
                    hg1.wait()
                    hc1 = pltpu.async_copy(r1b,
                                           agg_s.at[db[bb].at[2 * j + 1]],
                                           ss1, add=True)
                else:
                    hc0 = pltpu.async_copy(r0b, agg_s.at[db[bb].at[2 * j]],
                                           ss0, add=True)
                    hc1 = pltpu.async_copy(r0b,
                                           agg_s.at[db[bb].at[2 * j + 1]],
                                           ss1, add=True)
                hc0.wait()
                hc1.wait()
                return 0
            lax.fori_loop(0, BCH // 2, super2, 0)
            if gather:
                pltpu.async_copy(x_hbm.at[sb[bb].at[BCH - 1]], r0b,
                                 gs0).wait()
            pltpu.sync_copy(r0b, agg_s.at[db[bb].at[BCH - 1]], add=True)

        plsc.subcore_barrier()
        _copy_acc_out(c, s, agg_s, dest)

    if deg_out is not None:
        _fill(r1b, 0.0)
        _zero_acc(s, r1b, agg_s)
        _fill(r0b, 1.0)
        plsc.subcore_barrier()
        do_pass(False, deg_out)

    _fill(r1b, 0.0)
    _zero_acc(s, r1b, agg_s)
    plsc.subcore_barrier()
    do_pass(True, agg_out)


_SEG_SCRATCH = [
    pltpu.VMEM((BCH, K), jnp.int32),
    pltpu.VMEM((BCH, K), jnp.int32),
    pltpu.VMEM((BCH, K), jnp.int32),
    pltpu.VMEM((BCH, K), jnp.int32),
    pltpu.VMEM((K, D), jnp.float32),
    pltpu.VMEM((K, D), jnp.float32),
    pltpu.VMEM_SHARED((N, D), jnp.float32),
] + [pltpu.SemaphoreType.DMA] * 8


@functools.partial(
    pl.kernel,
    out_type=jax.ShapeDtypeStruct((NC, N, D), jnp.float32),
    mesh=_mesh,
    scratch_types=list(_SEG_SCRATCH),
)
def _segsum_sc(x_hbm, src_hbm, dst_hbm, agg_out, *scratch):
    _seg_body(x_hbm, src_hbm, dst_hbm, agg_out, *scratch)


@functools.partial(
    pl.kernel,
    out_type=[jax.ShapeDtypeStruct((NC, N, D), jnp.float32),
              jax.ShapeDtypeStruct((NC, N, D), jnp.float32)],
    mesh=_mesh,
    scratch_types=list(_SEG_SCRATCH),
)
def _segsum_deg_sc(x_hbm, src_hbm, dst_hbm, agg_out, deg_out, *scratch):
    _seg_body(x_hbm, src_hbm, dst_hbm, agg_out, *scratch, deg_out=deg_out)


BN = 400


def _mid_body(x, a0, a1, d0, d1, w1s, wl, w1n, b1, bl, w2n, w2s, b2,
              y_out, zs_out):
    deg = jnp.maximum(d0[...] + d1[...], 1.0)[:, 0:1]
    mean1 = (a0[...] + a1[...]) / deg
    h = jnp.dot(mean1, w1n[...], preferred_element_type=jnp.float32)
    h += jnp.dot(x[...], w1s[...] + wl[...], preferred_element_type=jnp.float32)
    z = jnp.maximum(h + (b1[...] + bl[...]), 0.0)
    y_out[...] = jnp.dot(z, w2n[...], preferred_element_type=jnp.float32)
    zs_out[...] = (jnp.dot(z, w2s[...], preferred_element_type=jnp.float32)
                   + b2[...])


def _mid_tc(x, a0, a1, d0, d1, W1s, Wl, W1n, b1, bl, W2n, W2s, b2):
    HID = W1n.shape[1]
    LAT = W2n.shape[1]
    grid = N // BN
    row = lambda i: (i, 0)
    rep = lambda i: (0, 0)
    return pl.pallas_call(
        _mid_body,
        grid=(grid,),
        in_specs=[
            pl.BlockSpec((BN, D), row),
            pl.BlockSpec((BN, D), row),
            pl.BlockSpec((BN, D), row),
            pl.BlockSpec((BN, D), row),
            pl.BlockSpec((BN, D), row),
            pl.BlockSpec((D, HID), rep),
            pl.BlockSpec((D, HID), rep),
            pl.BlockSpec((D, HID), rep),
            pl.BlockSpec((1, HID), rep),
            pl.BlockSpec((1, HID), rep),
            pl.BlockSpec((HID, LAT), rep),
            pl.BlockSpec((HID, LAT), rep),
            pl.BlockSpec((1, LAT), rep),
        ],
        out_specs=[
            pl.BlockSpec((BN, LAT), row),
            pl.BlockSpec((BN, LAT), row),
        ],
        out_shape=[
            jax.ShapeDtypeStruct((N, LAT), jnp.float32),
            jax.ShapeDtypeStruct((N, LAT), jnp.float32),
        ],
    )(x, a0, a1, d0, d1, W1s, Wl, W1n, b1, bl, W2n, W2s, b2)


def _fin_body(a0, a1, d0, d1, zs, out):
    deg = jnp.maximum(d0[...] + d1[...], 1.0)[:, 0:1]
    out[...] = (a0[...] + a1[...]) / deg + zs[...]


def _fin_tc(a0, a1, d0, d1, zs):
    grid = N // BN
    row = lambda i: (i, 0)
    return pl.pallas_call(
        _fin_body,
        grid=(grid,),
        in_specs=[
            pl.BlockSpec((BN, D), row),
            pl.BlockSpec((BN, D), row),
            pl.BlockSpec((BN, D), row),
            pl.BlockSpec((BN, D), row),
            pl.BlockSpec((BN, D), row),
        ],
        out_specs=pl.BlockSpec((BN, D), row),
        out_shape=jax.ShapeDtypeStruct((N, D), jnp.float32),
    )(a0, a1, d0, d1, zs)


def kernel(x, edge_index, W1n, W1s, b1, Wl, bl, W2n, W2s, b2):
    src = edge_index[0].reshape(NW, NB, BCH, K)
    dst = edge_index[1].reshape(NW, NB, BCH, K)
    agg1, degp = _segsum_deg_sc(x, src, dst)
    y, zs = _mid_tc(x, agg1[0], agg1[1], degp[0], degp[1],
                    W1s, Wl, W1n, b1.reshape(1, -1), bl.reshape(1, -1),
                    W2n, W2s, b2.reshape(1, -1))
    agg2 = _segsum_sc(y, src, dst)
    out = _fin_tc(agg2[0], agg2[1], degp[0], degp[1], zs)
    return out


# speedup vs baseline: 10.8570x; 1.1765x over previous
import functools

import jax
import jax.numpy as jnp
from jax import lax
from jax.experimental import pallas as pl
from jax.experimental.pallas import tpu as pltpu
from jax.experimental.pallas import tpu_sc as plsc

N = 10000
D = 128
E = 320000

NC = 2
NS = 16
NW = NC * NS
EPT = E // NW
K = 80
NCHUNK = EPT // K
RPT = 624
ZR = 104

_mesh = plsc.VectorSubcoreMesh(core_axis_name="c", subcore_axis_name="s")


def _fill(buf, val):
    def f(k, _):
        buf[k // 8, pl.ds((k % 8) * 16, 16)] = jnp.full((16,), val,
                                                        jnp.float32)
        return 0
    lax.fori_loop(0, K * 8, f, 0)


def _zero_acc(s, zbuf, agg_s):
    r0 = s * RPT
    for t in range(RPT // K):
        pltpu.sync_copy(zbuf, agg_s.at[pl.ds(r0 + t * K, K)])
    rem = RPT - (RPT // K) * K
    if rem:
        pltpu.sync_copy(zbuf.at[pl.ds(0, rem)],
                        agg_s.at[pl.ds(r0 + RPT - rem, rem)])

    @pl.when(s == NS - 1)
    def _():
        pltpu.sync_copy(zbuf.at[pl.ds(0, 16)], agg_s.at[pl.ds(N - 16, 16)])


def _copy_acc_out(c, s, agg_s, dest):
    r0 = s * RPT
    pltpu.sync_copy(agg_s.at[pl.ds(r0, RPT)], dest.at[c, pl.ds(r0, RPT)])

    @pl.when(s == NS - 1)
    def _():
        pltpu.sync_copy(agg_s.at[pl.ds(N - 16, 16)],
                        dest.at[c, pl.ds(N - 16, 16)])


NB = 5
BCH = NCHUNK // NB


def _seg_body(x_hbm, src_hbm, dst_hbm, agg_out,
              sb0, sb1, db0, db1, r0b, r1b, agg_s,
              gs0, gs1, ss0, ss1, lb0, lb1, lb2, lb3, deg_out=None):
    c = lax.axis_index("c")
    s = lax.axis_index("s")
    wid = c * NS + s
    sb = (sb0, sb1)
    db = (db0, db1)
    lsem = ((lb0, lb1), (lb2, lb3))

    def do_pass(gather, dest):
        if gather:
            pltpu.async_copy(src_hbm.at[wid, 0], sb[0], lsem[0][0])
        pltpu.async_copy(dst_hbm.at[wid, 0], db[0], lsem[0][1])
        for u in range(NB):
            bb = u & 1
            if gather:
                pltpu.make_async_copy(src_hbm.at[wid, 0], sb[bb],
                                      lsem[bb][0]).wait()
            pltpu.make_async_copy(dst_hbm.at[wid, 0], db[bb],
                                  lsem[bb][1]).wait()
            if u + 1 < NB:
                if gather:
                    pltpu.async_copy(src_hbm.at[wid, u + 1], sb[1 - bb],
                                     lsem[1 - bb][0])
                pltpu.async_copy(dst_hbm.at[wid, u + 1], db[1 - bb],
                                 lsem[1 - bb][1])

            if gather:
                def wait_g0():
                    pltpu.make_async_copy(x_hbm.at[sb[bb].at[0]], r0b,
                                          gs0).wait()

                def wait_g1():
                    pltpu.make_async_copy(x_hbm.at[sb[bb].at[0]], r1b,
                                          gs1).wait()

                def wait_c0():
                    pltpu.make_async_copy(r0b, agg_s.at[db[bb].at[0]],
                                          ss0).wait()

                def wait_c1():
                    pltpu.make_async_copy(r1b, agg_s.at[db[bb].at[0]],
                                          ss1).wait()

                pltpu.async_copy(x_hbm.at[sb[bb].at[0]], r0b, gs0)
                pltpu.async_copy(x_hbm.at[sb[bb].at[1]], r1b, gs1)

                def sup(j, _, bb=bb):
                    wait_g0()
                    pltpu.async_copy(r0b, agg_s.at[db[bb].at[2 * j]], ss0,
                                     add=True)
                    wait_c0()
                    pltpu.async_copy(x_hbm.at[sb[bb].at[2 * j + 2]], r0b,
                                     gs0)
                    wait_g1()
                    pltpu.async_copy(r1b, agg_s.at[db[bb].at[2 * j + 1]],
                                     ss1, add=True)
                    wait_c1()
                    pltpu.async_copy(x_hbm.at[sb[bb].at[2 * j + 3]], r1b,
                                     gs1)
                    return 0
                lax.fori_loop(0, (BCH - 3) // 2, sup, 0)
                wait_g0()
                pltpu.async_copy(r0b, agg_s.at[db[bb].at[BCH - 3]], ss0,
                                 add=True)
                wait_c0()
                pltpu.async_copy(x_hbm.at[sb[bb].at[BCH - 1]], r0b, gs0)
                wait_g1()
                pltpu.async_copy(r1b, agg_s.at[db[bb].at[BCH - 2]], ss1,
                                 add=True)
                wait_c1()
                wait_g0()
                pltpu.async_copy(r0b, agg_s.at[db[bb].at[BCH - 1]], ss0,
                                 add=True)
                wait_c0()
            else:
                def dfire(j, _, bb=bb):
                    pltpu.async_copy(r0b, agg_s.at[db[bb].at[j]], ss0,
                                     add=True)
                    return 0
                lax.fori_loop(0, BCH, dfire, 0)

                def ddrain(j, _, bb=bb):
                    pltpu.make_async_copy(r0b, agg_s.at[db[bb].at[0]],
                                          ss0).wait()
                    return 0
                lax.fori_loop(0, BCH, ddrain, 0)

        plsc.subcore_barrier()
        _copy_acc_out(c, s, agg_s, dest)

    if deg_out is not None:
        _fill(r1b, 0.0)
        _zero_acc(s, r1b, agg_s)
        _fill(r0b, 1.0)
        plsc.subcore_barrier()
        do_pass(False, deg_out)

    _fill(r1b, 0.0)
    _zero_acc(s, r1b, agg_s)
    plsc.subcore_barrier()
    do_pass(True, agg_out)


_SEG_SCRATCH = [
    pltpu.VMEM((BCH, K), jnp.int32),
    pltpu.VMEM((BCH, K), jnp.int32),
    pltpu.VMEM((BCH, K), jnp.int32),
    pltpu.VMEM((BCH, K), jnp.int32),
    pltpu.VMEM((K, D), jnp.float32),
    pltpu.VMEM((K, D), jnp.float32),
    pltpu.VMEM_SHARED((N, D), jnp.float32),
] + [pltpu.SemaphoreType.DMA] * 8


@functools.partial(
    pl.kernel,
    out_type=jax.ShapeDtypeStruct((NC, N, D), jnp.float32),
    mesh=_mesh,
    scratch_types=list(_SEG_SCRATCH),
)
def _segsum_sc(x_hbm, src_hbm, dst_hbm, agg_out, *scratch):
    _seg_body(x_hbm, src_hbm, dst_hbm, agg_out, *scratch)


@functools.partial(
    pl.kernel,
    out_type=[jax.ShapeDtypeStruct((NC, N, D), jnp.float32),
              jax.ShapeDtypeStruct((NC, N, D), jnp.float32)],
    mesh=_mesh,
    scratch_types=list(_SEG_SCRATCH),
)
def _segsum_deg_sc(x_hbm, src_hbm, dst_hbm, agg_out, deg_out, *scratch):
    _seg_body(x_hbm, src_hbm, dst_hbm, agg_out, *scratch, deg_out=deg_out)


BN = 400


def _mid_body(x, a0, a1, d0, d1, w1s, wl, w1n, b1, bl, w2n, w2s, b2,
              y_out, zs_out):
    deg = jnp.maximum(d0[...] + d1[...], 1.0)[:, 0:1]
    mean1 = (a0[...] + a1[...]) / deg
    h = jnp.dot(mean1, w1n[...], preferred_element_type=jnp.float32)
    h += jnp.dot(x[...], w1s[...] + wl[...], preferred_element_type=jnp.float32)
    z = jnp.maximum(h + (b1[...] + bl[...]), 0.0)
    y_out[...] = jnp.dot(z, w2n[...], preferred_element_type=jnp.float32)
    zs_out[...] = (jnp.dot(z, w2s[...], preferred_element_type=jnp.float32)
                   + b2[...])


def _mid_tc(x, a0, a1, d0, d1, W1s, Wl, W1n, b1, bl, W2n, W2s, b2):
    HID = W1n.shape[1]
    LAT = W2n.shape[1]
    grid = N // BN
    row = lambda i: (i, 0)
    rep = lambda i: (0, 0)
    return pl.pallas_call(
        _mid_body,
        grid=(grid,),
        in_specs=[
            pl.BlockSpec((BN, D), row),
            pl.BlockSpec((BN, D), row),
            pl.BlockSpec((BN, D), row),
            pl.BlockSpec((BN, D), row),
            pl.BlockSpec((BN, D), row),
            pl.BlockSpec((D, HID), rep),
            pl.BlockSpec((D, HID), rep),
            pl.BlockSpec((D, HID), rep),
            pl.BlockSpec((1, HID), rep),
            pl.BlockSpec((1, HID), rep),
            pl.BlockSpec((HID, LAT), rep),
            pl.BlockSpec((HID, LAT), rep),
            pl.BlockSpec((1, LAT), rep),
        ],
        out_specs=[
            pl.BlockSpec((BN, LAT), row),
            pl.BlockSpec((BN, LAT), row),
        ],
        out_shape=[
            jax.ShapeDtypeStruct((N, LAT), jnp.float32),
            jax.ShapeDtypeStruct((N, LAT), jnp.float32),
        ],
    )(x, a0, a1, d0, d1, W1s, Wl, W1n, b1, bl, W2n, W2s, b2)


def _fin_body(a0, a1, d0, d1, zs, out):
    deg = jnp.maximum(d0[...] + d1[...], 1.0)[:, 0:1]
    out[...] = (a0[...] + a1[...]) / deg + zs[...]


def _fin_tc(a0, a1, d0, d1, zs):
    grid = N // BN
    row = lambda i: (i, 0)
    return pl.pallas_call(
        _fin_body,
        grid=(grid,),
        in_specs=[
            pl.BlockSpec((BN, D), row),
            pl.BlockSpec((BN, D), row),
            pl.BlockSpec((BN, D), row),
            pl.BlockSpec((BN, D), row),
            pl.BlockSpec((BN, D), row),
        ],
        out_specs=pl.BlockSpec((BN, D), row),
        out_shape=jax.ShapeDtypeStruct((N, D), jnp.float32),
    )(a0, a1, d0, d1, zs)


def kernel(x, edge_index, W1n, W1s, b1, Wl, bl, W2n, W2s, b2):
    src = edge_index[0].reshape(NW, NB, BCH, K)
    dst = edge_index[1].reshape(NW, NB, BCH, K)
    agg1, degp = _segsum_deg_sc(x, src, dst)
    y, zs = _mid_tc(x, agg1[0], agg1[1], degp[0], degp[1],
                    W1s, Wl, W1n, b1.reshape(1, -1), bl.reshape(1, -1),
                    W2n, W2s, b2.reshape(1, -1))
    agg2 = _segsum_sc(y, src, dst)
    out = _fin_tc(agg2[0], agg2[1], degp[0], degp[1], zs)
    return out


# speedup vs baseline: 10.8580x; 1.0001x over previous
import functools

import jax
import jax.numpy as jnp
from jax import lax
from jax.experimental import pallas as pl
from jax.experimental.pallas import tpu as pltpu
from jax.experimental.pallas import tpu_sc as plsc

N = 10000
D = 128
E = 320000

NC = 2
NS = 16
NW = NC * NS
EPT = E // NW
K = 80
NCHUNK = EPT // K
RPT = 624

_mesh = plsc.VectorSubcoreMesh(core_axis_name="c", subcore_axis_name="s")


def _fill(buf, val):
    def f(k, _):
        buf[k // 8, pl.ds((k % 8) * 16, 16)] = jnp.full((16,), val,
                                                        jnp.float32)
        return 0
    lax.fori_loop(0, K * 8, f, 0)


def _zero_acc(s, zbuf, agg_s):
    r0 = s * RPT
    for t in range(RPT // K):
        pltpu.sync_copy(zbuf, agg_s.at[pl.ds(r0 + t * K, K)])
    rem = RPT - (RPT // K) * K
    if rem:
        pltpu.sync_copy(zbuf.at[pl.ds(0, rem)],
                        agg_s.at[pl.ds(r0 + RPT - rem, rem)])

    @pl.when(s == NS - 1)
    def _():
        pltpu.sync_copy(zbuf.at[pl.ds(0, 16)], agg_s.at[pl.ds(N - 16, 16)])


def _copy_acc_out(c, s, agg_s, dest):
    r0 = s * RPT
    pltpu.sync_copy(agg_s.at[pl.ds(r0, RPT)], dest.at[c, pl.ds(r0, RPT)])

    @pl.when(s == NS - 1)
    def _():
        pltpu.sync_copy(agg_s.at[pl.ds(N - 16, 16)],
                        dest.at[c, pl.ds(N - 16, 16)])


NB = 5
BCH = NCHUNK // NB


def _seg_body(x_hbm, src_hbm, dst_hbm, agg_out,
              sb0, sb1, db0, db1, r0b, r1b, agg_s,
              gs0, gs1, ss0, ss1, lb0, lb1, lb2, lb3, deg_out=None):
    c = lax.axis_index("c")
    s = lax.axis_index("s")
    wid = c * NS + s
    sb = (sb0, sb1)
    db = (db0, db1)
    lsem = ((lb0, lb1), (lb2, lb3))

    def do_pass(gather, dest):
        if gather:
            pltpu.async_copy(src_hbm.at[wid, 0], sb[0], lsem[0][0])
        pltpu.async_copy(dst_hbm.at[wid, 0], db[0], lsem[0][1])
        for u in range(NB):
            bb = u & 1
            if gather:
                pltpu.make_async_copy(src_hbm.at[wid, 0], sb[bb],
                                      lsem[bb][0]).wait()
            pltpu.make_async_copy(dst_hbm.at[wid, 0], db[bb],
                                  lsem[bb][1]).wait()
            if u + 1 < NB:
                if gather:
                    pltpu.async_copy(src_hbm.at[wid, u + 1], sb[1 - bb],
                                     lsem[1 - bb][0])
                pltpu.async_copy(dst_hbm.at[wid, u + 1], db[1 - bb],
                                 lsem[1 - bb][1])

            if gather:
                def wait_g0():
                    pltpu.make_async_copy(x_hbm.at[sb[bb].at[0]], r0b,
                                          gs0).wait()

                def wait_g1():
                    pltpu.make_async_copy(x_hbm.at[sb[bb].at[0]], r1b,
                                          gs1).wait()

                def wait_c0():
                    pltpu.make_async_copy(r0b, agg_s.at[db[bb].at[0]],
                                          ss0).wait()

                def wait_c1():
                    pltpu.make_async_copy(r1b, agg_s.at[db[bb].at[0]],
                                          ss1).wait()

                pltpu.async_copy(x_hbm.at[sb[bb].at[0]], r0b, gs0)
                pltpu.async_copy(x_hbm.at[sb[bb].at[1]], r1b, gs1)

                def sup(j, _, bb=bb):
                    wait_g0()
                    pltpu.async_copy(r0b, agg_s.at[db[bb].at[2 * j]], ss0,
                                     add=True)
                    wait_c0()
                    pltpu.async_copy(x_hbm.at[sb[bb].at[2 * j + 2]], r0b,
                                     gs0)
                    wait_g1()
                    pltpu.async_copy(r1b, agg_s.at[db[bb].at[2 * j + 1]],
                                     ss1, add=True)
                    wait_c1()
                    pltpu.async_copy(x_hbm.at[sb[bb].at[2 * j + 3]], r1b,
                                     gs1)
                    return 0
                lax.fori_loop(0, (BCH - 3) // 2, sup, 0)
                wait_g0()
                pltpu.async_copy(r0b, agg_s.at[db[bb].at[BCH - 3]], ss0,
                                 add=True)
                wait_c0()
                pltpu.async_copy(x_hbm.at[sb[bb].at[BCH - 1]], r0b, gs0)
                wait_g1()
                pltpu.async_copy(r1b, agg_s.at[db[bb].at[BCH - 2]], ss1,
                                 add=True)
                wait_c1()
                wait_g0()
                pltpu.async_copy(r0b, agg_s.at[db[bb].at[BCH - 1]], ss0,
                                 add=True)
                wait_c0()
            else:
                def dfire(j, _, bb=bb):
                    pltpu.async_copy(r0b, agg_s.at[db[bb].at[j]], ss0,
                                     add=True)
                    return 0
                lax.fori_loop(0, BCH, dfire, 0)

                def ddrain(j, _, bb=bb):
                    pltpu.make_async_copy(r0b, agg_s.at[db[bb].at[0]],
                                          ss0).wait()
                    return 0
                lax.fori_loop(0, BCH, ddrain, 0)

        plsc.subcore_barrier()
        _copy_acc_out(c, s, agg_s, dest)

    if deg_out is not None:
        _fill(r1b, 0.0)
        _zero_acc(s, r1b, agg_s)
        _fill(r0b, 1.0)
        plsc.subcore_barrier()
        do_pass(False, deg_out)

    _fill(r1b, 0.0)
    _zero_acc(s, r1b, agg_s)
    plsc.subcore_barrier()
    do_pass(True, agg_out)


_SEG_SCRATCH = [
    pltpu.VMEM((BCH, K), jnp.int32),
    pltpu.VMEM((BCH, K), jnp.int32),
    pltpu.VMEM((BCH, K), jnp.int32),
    pltpu.VMEM((BCH, K), jnp.int32),
    pltpu.VMEM((K, D), jnp.float32),
    pltpu.VMEM((K, D), jnp.float32),
    pltpu.VMEM_SHARED((N, D), jnp.float32),
] + [pltpu.SemaphoreType.DMA] * 8


@functools.partial(
    pl.kernel,
    out_type=jax.ShapeDtypeStruct((NC, N, D), jnp.float32),
    mesh=_mesh,
    scratch_types=list(_SEG_SCRATCH),
)
def _segsum_sc(x_hbm, src_hbm, dst_hbm, agg_out, *scratch):
    _seg_body(x_hbm, src_hbm, dst_hbm, agg_out, *scratch)


@functools.partial(
    pl.kernel,
    out_type=[jax.ShapeDtypeStruct((NC, N, D), jnp.float32),
              jax.ShapeDtypeStruct((NC, N, D), jnp.float32)],
    mesh=_mesh,
    scratch_types=list(_SEG_SCRATCH),
)
def _segsum_deg_sc(x_hbm, src_hbm, dst_hbm, agg_out, deg_out, *scratch):
    _seg_body(x_hbm, src_hbm, dst_hbm, agg_out, *scratch, deg_out=deg_out)


BN = 400


def _mid_body(x, a0, a1, d0, d1, w1s, wl, w1n, b1, bl, w2n, w2s, b2,
              y_out, zs_out):
    deg = jnp.maximum(d0[...] + d1[...], 1.0)[:, 0:1]
    mean1 = (a0[...] + a1[...]) / deg
    h = jnp.dot(mean1, w1n[...], preferred_element_type=jnp.float32)
    h += jnp.dot(x[...], w1s[...] + wl[...], preferred_element_type=jnp.float32)
    z = jnp.maximum(h + (b1[...] + bl[...]), 0.0)
    y_out[...] = jnp.dot(z, w2n[...], preferred_element_type=jnp.float32)
    zs_out[...] = (jnp.dot(z, w2s[...], preferred_element_type=jnp.float32)
                   + b2[...])


def _mid_tc(x, a0, a1, d0, d1, W1s, Wl, W1n, b1, bl, W2n, W2s, b2):
    HID = W1n.shape[1]
    LAT = W2n.shape[1]
    grid = N // BN
    row = lambda i: (i, 0)
    rep = lambda i: (0, 0)
    return pl.pallas_call(
        _mid_body,
        grid=(grid,),
        in_specs=[
            pl.BlockSpec((BN, D), row),
            pl.BlockSpec((BN, D), row),
            pl.BlockSpec((BN, D), row),
            pl.BlockSpec((BN, D), row),
            pl.BlockSpec((BN, D), row),
            pl.BlockSpec((D, HID), rep),
            pl.BlockSpec((D, HID), rep),
            pl.BlockSpec((D, HID), rep),
            pl.BlockSpec((1, HID), rep),
            pl.BlockSpec((1, HID), rep),
            pl.BlockSpec((HID, LAT), rep),
            pl.BlockSpec((HID, LAT), rep),
            pl.BlockSpec((1, LAT), rep),
        ],
        out_specs=[
            pl.BlockSpec((BN, LAT), row),
            pl.BlockSpec((BN, LAT), row),
        ],
        out_shape=[
            jax.ShapeDtypeStruct((N, LAT), jnp.float32),
            jax.ShapeDtypeStruct((N, LAT), jnp.float32),
        ],
    )(x, a0, a1, d0, d1, W1s, Wl, W1n, b1, bl, W2n, W2s, b2)


def _fin_body(a0, a1, d0, d1, zs, out):
    deg = jnp.maximum(d0[...] + d1[...], 1.0)[:, 0:1]
    out[...] = (a0[...] + a1[...]) / deg + zs[...]


def _fin_tc(a0, a1, d0, d1, zs):
    grid = N // BN
    row = lambda i: (i, 0)
    return pl.pallas_call(
        _fin_body,
        grid=(grid,),
        in_specs=[
            pl.BlockSpec((BN, D), row),
            pl.BlockSpec((BN, D), row),
            pl.BlockSpec((BN, D), row),
            pl.BlockSpec((BN, D), row),
            pl.BlockSpec((BN, D), row),
        ],
        out_specs=pl.BlockSpec((BN, D), row),
        out_shape=jax.ShapeDtypeStruct((N, D), jnp.float32),
    )(a0, a1, d0, d1, zs)


def kernel(x, edge_index, W1n, W1s, b1, Wl, bl, W2n, W2s, b2):
    src = edge_index[0].reshape(NW, NB, BCH, K)
    dst = edge_index[1].reshape(NW, NB, BCH, K)
    agg1, degp = _segsum_deg_sc(x, src, dst)
    y, zs = _mid_tc(x, agg1[0], agg1[1], degp[0], degp[1],
                    W1s, Wl, W1n, b1.reshape(1, -1), bl.reshape(1, -1),
                    W2n, W2s, b2.reshape(1, -1))
    agg2 = _segsum_sc(y, src, dst)
    out = _fin_tc(agg2[0], agg2[1], degp[0], degp[1], zs)
    return out
